# baseline (speedup 1.0000x reference)
import functools

import jax
import jax.numpy as jnp
from jax import lax
from jax.experimental import pallas as pl
from jax.experimental.pallas import tpu as pltpu
from jax.experimental.pallas import tpu_sc as plsc

N = 10000
E = 320000
D = 128
R = 4

NC = 2
NS = 16
NW = NC * NS

CHUNK = 88
EPB = 240
NW_E = 16
EPT = EPB * CHUNK
E_PAD = NW_E * EPT
EPT0 = E_PAD // NW
NPAD = 10240
RPT = N // NS

_NBUF = 4
_NIDX = 8
_SCATTER = True

_BLK = 1000


def _mesh():
    return plsc.VectorSubcoreMesh(
        core_axis_name="c", subcore_axis_name="s", num_cores=NC, num_subcores=NS
    )


_SC_PARAMS = dict(
    compiler_params=pltpu.CompilerParams(
        needs_layout_passes=False, use_tc_tiling_on_sc=False
    ),
)


def _sck0_body(col_h, et_h, gidx_h, degp_h, colv, etv, gv, degv):
    c = lax.axis_index("c")
    s = lax.axis_index("s")
    w = c * NS + s
    pltpu.sync_copy(col_h.at[w], colv)
    pltpu.sync_copy(et_h.at[w], etv)

    zero16 = jnp.zeros((16,), jnp.float32)

    def _z(i, carry):
        degv[pl.ds(i * 16, 16)] = zero16
        return carry

    lax.fori_loop(0, NPAD // 16, _z, 0)

    ones16 = jnp.ones((16,), jnp.float32)

    def _e(k, carry):
        cc = colv[pl.ds(k * 16, 16)]
        tt = etv[pl.ds(k * 16, 16)]
        gv[pl.ds(k * 16, 16)] = tt * N + cc
        plsc.addupdate_scatter(degv, [cc], ones16)
        return carry

    lax.fori_loop(0, EPT0 // 16, _e, 0)

    pltpu.sync_copy(gv, gidx_h.at[w])
    pltpu.sync_copy(degv, degp_h.at[w])


def _sck0(col2, et2):
    return pl.kernel(
        _sck0_body,
        out_type=(
            jax.ShapeDtypeStruct((NW, EPT0), jnp.int32),
            jax.ShapeDtypeStruct((NW, NPAD), jnp.float32),
        ),
        mesh=_mesh(),
        scratch_types=[
            pltpu.VMEM((EPT0,), jnp.int32),
            pltpu.VMEM((EPT0,), jnp.int32),
            pltpu.VMEM((EPT0,), jnp.int32),
            pltpu.VMEM((NPAD,), jnp.float32),
        ],
        **_SC_PARAMS,
    )(col2, et2)


def _sck_edge_body(hp_h, comb_h, p_h,
                   gb0, gb1, gb2, gb3,
                   is0, is1, is2, is3, is4, is5, is6, is7,
                   acc,
                   gs0, gs1, gs2, gs3, ss0, ss1, ss2, ss3,
                   im0, im1, im2, im3, im4, im5, im6, im7):
    c = lax.axis_index("c")
    s = lax.axis_index("s")
    gb = [gb0, gb1, gb2, gb3]
    isl = [is0, is1, is2, is3, is4, is5, is6, is7]
    gsem = [gs0, gs1, gs2, gs3]
    ssem = [ss0, ss1, ss2, ss3]
    isem = [im0, im1, im2, im3, im4, im5, im6, im7]

    @pl.when(c == 1)
    def _sc0_work():
        _sck_edge_core(hp_h, comb_h, p_h, gb, isl, acc, gsem, ssem, isem, s)


def _sck_edge_core(hp_h, comb_h, p_h, gb, isl, acc, gsem, ssem, isem, s):
    gb0 = gb[0]
    w = s

    for k in range(4):
        pltpu.async_copy(comb_h.at[w, k], isl[k], isem[k])

    zero16 = jnp.zeros((16,), jnp.float32)

    def _zrow(i, carry):
        for q in range(D // 16):
            gb0[i, pl.ds(q * 16, 16)] = zero16
        return carry

    lax.fori_loop(0, CHUNK, _zrow, 0)

    base = s * RPT
    for k in range(RPT // CHUNK):
        pltpu.sync_copy(gb0, acc.at[pl.ds(base + k * CHUNK, CHUNK)])
    rem = RPT % CHUNK
    if rem:
        pltpu.sync_copy(gb0.at[pl.ds(0, rem)],
                        acc.at[pl.ds(base + RPT - rem, rem)])

    for k in range(2):
        pltpu.make_async_copy(comb_h.at[w, k], isl[k], isem[k]).wait()
        pltpu.async_copy(hp_h.at[isl[k].at[0]], gb[k], gsem[k])
    plsc.subcore_barrier()

    def _iter(j, db):
        bi = db % _NBUF
        ii = db % _NIDX
        bn = (db + 2) % _NBUF
        inx = (db + 2) % _NIDX
        ipf = (db + 4) % _NIDX

        pltpu.make_async_copy(hp_h.at[isl[ii].at[0]], gb[bi], gsem[bi]).wait()

        if _SCATTER:
            @pl.when(j >= 2)
            def _drain():
                pltpu.make_async_copy(
                    gb[bn], acc.at[isl[(db - 2) % _NIDX].at[1]], ssem[bn]
                ).wait()

        @pl.when(j + 2 < EPB)
        def _gather_ahead():
            pltpu.make_async_copy(comb_h.at[w, j + 2], isl[inx], isem[inx]).wait()
            pltpu.async_copy(hp_h.at[isl[inx].at[0]], gb[bn], gsem[bn])

        @pl.when(j + 4 < EPB)
        def _idx_ahead():
            pltpu.async_copy(comb_h.at[w, j + 4], isl[ipf], isem[ipf])

        def _srow(i, carry2):
            abits = plsc.load_gather(
                isl[ii], [jnp.full((16,), 2, jnp.int32), jnp.full((16,), i, jnp.int32)]
            )
            a = plsc.bitcast(abits, jnp.float32)
            for q in range(D // 16):
                gb[bi][i, pl.ds(q * 16, 16)] = gb[bi][i, pl.ds(q * 16, 16)] * a
            return carry2

        lax.fori_loop(0, CHUNK, _srow, 0, unroll=4)

        if _SCATTER:
            pltpu.async_copy(gb[bi], acc.at[isl[ii].at[1]], ssem[bi], add=True)

    def _outer(t, carry):
        for db in range(_NIDX):
            _iter(t * _NIDX + db, db)
        return carry

    lax.fori_loop(0, EPB // _NIDX, _outer, 0)

    for j in (EPB - 2, EPB - 1) if _SCATTER else ():
        pltpu.make_async_copy(
            gb[j % _NBUF], acc.at[isl[j % _NIDX].at[1]], ssem[j % _NBUF]
        ).wait()

    plsc.subcore_barrier()
    pltpu.sync_copy(acc.at[pl.ds(base, RPT)], p_h.at[pl.ds(base, RPT)])


def _sck_edge(hp_flat, comb):
    return pl.kernel(
        _sck_edge_body,
        out_type=jax.ShapeDtypeStruct((N, D), jnp.float32),
        mesh=_mesh(),
        scratch_types=(
            [pltpu.VMEM((CHUNK, D), jnp.float32)] * _NBUF
            + [pltpu.VMEM((3, CHUNK), jnp.int32)] * _NIDX
            + [pltpu.VMEM_SHARED((N, D), jnp.float32)]
            + [pltpu.SemaphoreType.DMA] * (2 * _NBUF + _NIDX)
        ),
        **_SC_PARAMS,
    )(hp_flat, comb)


def _tck0_body(degp_ref, dis_ref):
    deg = jnp.sum(degp_ref[...], axis=0)
    dis = jnp.where(deg > 0, lax.rsqrt(deg), 0.0)
    dis_ref[...] = dis[:, None]


def _tck0(degp):
    return pl.pallas_call(
        _tck0_body,
        out_shape=jax.ShapeDtypeStruct((NPAD, 1), jnp.float32),
    )(degp)


def _tck1_body(x_ref, dis_ref, w_ref, hp_ref):
    xs = x_ref[...] * dis_ref[...]
    for r in range(R):
        hp_ref[r, :, :] = jnp.dot(xs, w_ref[r], preferred_element_type=jnp.float32)


def _tck1(x, dis, w):
    return pl.pallas_call(
        _tck1_body,
        grid=(N // _BLK,),
        in_specs=[
            pl.BlockSpec((_BLK, D), lambda i: (i, 0)),
            pl.BlockSpec((_BLK, 1), lambda i: (i, 0)),
            pl.BlockSpec((R, D, D), lambda i: (0, 0, 0)),
        ],
        out_specs=pl.BlockSpec((R, _BLK, D), lambda i: (0, i, 0)),
        out_shape=jax.ShapeDtypeStruct((R, N, D), jnp.float32),
    )(x, dis, w)


def _tck_mid_body(p_ref, dis_ref, b_ref, w_ref, z_ref, hp_ref):
    dis = dis_ref[...]
    v = p_ref[...] * dis + b_ref[...]
    z = jnp.where(v >= 0, v, 0.01 * v)
    z_ref[...] = z
    zs = z * dis
    for r in range(R):
        hp_ref[r, :, :] = jnp.dot(zs, w_ref[r], preferred_element_type=jnp.float32)


def _tck_mid(p, dis, b, w):
    return pl.pallas_call(
        _tck_mid_body,
        grid=(N // _BLK,),
        in_specs=[
            pl.BlockSpec((_BLK, D), lambda i: (i, 0)),
            pl.BlockSpec((_BLK, 1), lambda i: (i, 0)),
            pl.BlockSpec((1, D), lambda i: (0, 0)),
            pl.BlockSpec((R, D, D), lambda i: (0, 0, 0)),
        ],
        out_specs=[
            pl.BlockSpec((_BLK, D), lambda i: (i, 0)),
            pl.BlockSpec((R, _BLK, D), lambda i: (0, i, 0)),
        ],
        out_shape=[
            jax.ShapeDtypeStruct((N, D), jnp.float32),
            jax.ShapeDtypeStruct((R, N, D), jnp.float32),
        ],
    )(p, dis, b, w)


def _tck_fin_body(p_ref, dis_ref, b_ref, x_ref, z1_ref, z2_ref, z_ref):
    v = p_ref[...] * dis_ref[...] + b_ref[...]
    z3 = jnp.where(v >= 0, v, 0.01 * v)
    z_ref[...] = (x_ref[...] + z1_ref[...] + z2_ref[...] + z3) * 0.25


def _tck_fin(p, dis, b, x, z1, z2):
    return pl.pallas_call(
        _tck_fin_body,
        grid=(N // _BLK,),
        in_specs=[
            pl.BlockSpec((_BLK, D), lambda i: (i, 0)),
            pl.BlockSpec((_BLK, 1), lambda i: (i, 0)),
            pl.BlockSpec((1, D), lambda i: (0, 0)),
            pl.BlockSpec((_BLK, D), lambda i: (i, 0)),
            pl.BlockSpec((_BLK, D), lambda i: (i, 0)),
            pl.BlockSpec((_BLK, D), lambda i: (i, 0)),
        ],
        out_specs=pl.BlockSpec((_BLK, D), lambda i: (i, 0)),
        out_shape=jax.ShapeDtypeStruct((N, D), jnp.float32),
    )(p, dis, b, x, z1, z2)


def kernel(x, edge_index, edge_type, edge_attr, w1, b1, w2, b2, w3, b3):
    pad = E_PAD - E
    row2 = jnp.concatenate([edge_index[0], jnp.zeros((pad,), jnp.int32)])
    col2 = jnp.concatenate([edge_index[1], jnp.full((pad,), N, jnp.int32)]).reshape(NW, EPT0)
    et2 = jnp.concatenate([edge_type, jnp.zeros((pad,), jnp.int32)]).reshape(NW, EPT0)
    attr2 = jnp.concatenate([edge_attr, jnp.zeros((pad,), jnp.float32)])

    gidx2, degp = _sck0(col2, et2)

    comb = jnp.concatenate(
        [
            gidx2.reshape(NW_E, EPB, 1, CHUNK),
            row2.reshape(NW_E, EPB, 1, CHUNK),
            lax.bitcast_convert_type(attr2, jnp.int32).reshape(NW_E, EPB, 1, CHUNK),
        ],
        axis=2,
    )

    dis = _tck0(degp)

    hp1 = _tck1(x, dis, w1)
    p1 = _sck_edge(hp1.reshape(R * N, D), comb)
    z1, hp2 = _tck_mid(p1, dis, b1.reshape(1, D), w2)
    p2 = _sck_edge(hp2.reshape(R * N, D), comb)
    z2, hp3 = _tck_mid(p2, dis, b2.reshape(1, D), w3)
    p3 = _sck_edge(hp3.reshape(R * N, D), comb)
    z = _tck_fin(p3, dis, b3.reshape(1, D), x, z1, z2)
    return z


# speedup vs baseline: 2.6636x; 2.6636x over previous
import functools

import jax
import jax.numpy as jnp
from jax import lax
from jax.experimental import pallas as pl
from jax.experimental.pallas import tpu as pltpu
from jax.experimental.pallas import tpu_sc as plsc

N = 10000
E = 320000
D = 128
R = 4

NC = 2
NS = 16
NW = NC * NS

CHUNK = 128
EPB = 79
EPT = EPB * CHUNK
E_PAD = NW * EPT
NPAD = 10240
RPT = N // NS

_BLK = 1000


def _mesh():
    return plsc.VectorSubcoreMesh(
        core_axis_name="c", subcore_axis_name="s", num_cores=NC, num_subcores=NS
    )


_SC_PARAMS = dict(
    compiler_params=pltpu.CompilerParams(
        needs_layout_passes=False, use_tc_tiling_on_sc=False
    ),
)


def _sck0_body(col_h, et_h, gidx_h, degp_h, colv, etv, gv, degv):
    c = lax.axis_index("c")
    s = lax.axis_index("s")
    w = c * NS + s
    pltpu.sync_copy(col_h.at[w], colv)
    pltpu.sync_copy(et_h.at[w], etv)

    zero16 = jnp.zeros((16,), jnp.float32)

    def _z(i, carry):
        degv[pl.ds(i * 16, 16)] = zero16
        return carry

    lax.fori_loop(0, NPAD // 16, _z, 0)

    ones16 = jnp.ones((16,), jnp.float32)

    def _e(k, carry):
        cc = colv[pl.ds(k * 16, 16)]
        tt = etv[pl.ds(k * 16, 16)]
        gv[pl.ds(k * 16, 16)] = tt * N + cc
        plsc.addupdate_scatter(degv, [cc], ones16)
        return carry

    lax.fori_loop(0, EPT // 16, _e, 0)

    pltpu.sync_copy(gv, gidx_h.at[w])
    pltpu.sync_copy(degv, degp_h.at[w])


def _sck0(col2, et2):
    return pl.kernel(
        _sck0_body,
        out_type=(
            jax.ShapeDtypeStruct((NW, EPT), jnp.int32),
            jax.ShapeDtypeStruct((NW, NPAD), jnp.float32),
        ),
        mesh=_mesh(),
        scratch_types=[
            pltpu.VMEM((EPT,), jnp.int32),
            pltpu.VMEM((EPT,), jnp.int32),
            pltpu.VMEM((EPT,), jnp.int32),
            pltpu.VMEM((NPAD,), jnp.float32),
        ],
        **_SC_PARAMS,
    )(col2, et2)


def _sck_edge_body(hp_h, gidx_h, attr_h, row_h, p_h, gidxv, attrv, rowv, gbuf, acc, sem):
    c = lax.axis_index("c")
    s = lax.axis_index("s")
    w = c * NS + s
    pltpu.sync_copy(gidx_h.at[w], gidxv)
    pltpu.sync_copy(attr_h.at[w], attrv)
    pltpu.sync_copy(row_h.at[w], rowv)

    zero16 = jnp.zeros((16,), jnp.float32)

    def _zrow(i, carry):
        for q in range(D // 16):
            gbuf[i, pl.ds(q * 16, 16)] = zero16
        return carry

    lax.fori_loop(0, CHUNK, _zrow, 0)

    base = s * RPT
    for k in range(4):
        pltpu.sync_copy(gbuf, acc.at[pl.ds(base + k * CHUNK, CHUNK)])
    pltpu.sync_copy(gbuf.at[pl.ds(0, RPT - 4 * CHUNK)],
                    acc.at[pl.ds(base + 4 * CHUNK, RPT - 4 * CHUNK)])
    plsc.subcore_barrier()

    def _chunk(j, carry):
        pltpu.async_copy(hp_h.at[gidxv.at[j]], gbuf, sem).wait()

        def _srow(i, carry2):
            a = plsc.load_gather(attrv, [jnp.full((16,), j * CHUNK + i, jnp.int32)])
            for q in range(D // 16):
                gbuf[i, pl.ds(q * 16, 16)] = gbuf[i, pl.ds(q * 16, 16)] * a
            return carry2

        lax.fori_loop(0, CHUNK, _srow, 0, unroll=4)
        pltpu.sync_copy(gbuf, acc.at[rowv.at[j]], add=True)
        return carry

    lax.fori_loop(0, EPB, _chunk, 0)

    plsc.subcore_barrier()
    pltpu.sync_copy(acc.at[pl.ds(base, RPT)], p_h.at[c, pl.ds(base, RPT)])


def _sck_edge(hp_flat, gidx3, attr2, row3):
    return pl.kernel(
        _sck_edge_body,
        out_type=jax.ShapeDtypeStruct((NC, N, D), jnp.float32),
        mesh=_mesh(),
        scratch_types=[
            pltpu.VMEM((EPB, CHUNK), jnp.int32),
            pltpu.VMEM((EPT,), jnp.float32),
            pltpu.VMEM((EPB, CHUNK), jnp.int32),
            pltpu.VMEM((CHUNK, D), jnp.float32),
            pltpu.VMEM_SHARED((N, D), jnp.float32),
            pltpu.SemaphoreType.DMA,
        ],
        **_SC_PARAMS,
    )(hp_flat, gidx3, attr2, row3)


def _tck0_body(degp_ref, dis_ref):
    deg = jnp.sum(degp_ref[...], axis=0)
    dis = jnp.where(deg > 0, lax.rsqrt(deg), 0.0)
    dis_ref[...] = dis[:, None]


def _tck0(degp):
    return pl.pallas_call(
        _tck0_body,
        out_shape=jax.ShapeDtypeStruct((NPAD, 1), jnp.float32),
    )(degp)


def _tck1_body(x_ref, dis_ref, w_ref, hp_ref):
    xs = x_ref[...] * dis_ref[...]
    for r in range(R):
        hp_ref[r, :, :] = jnp.dot(xs, w_ref[r], preferred_element_type=jnp.float32)


def _tck1(x, dis, w):
    return pl.pallas_call(
        _tck1_body,
        grid=(N // _BLK,),
        in_specs=[
            pl.BlockSpec((_BLK, D), lambda i: (i, 0)),
            pl.BlockSpec((_BLK, 1), lambda i: (i, 0)),
            pl.BlockSpec((R, D, D), lambda i: (0, 0, 0)),
        ],
        out_specs=pl.BlockSpec((R, _BLK, D), lambda i: (0, i, 0)),
        out_shape=jax.ShapeDtypeStruct((R, N, D), jnp.float32),
    )(x, dis, w)


def _tck_mid_body(p_ref, dis_ref, b_ref, w_ref, z_ref, hp_ref):
    dis = dis_ref[...]
    v = (p_ref[0] + p_ref[1]) * dis + b_ref[...]
    z = jnp.where(v >= 0, v, 0.01 * v)
    z_ref[...] = z
    zs = z * dis
    for r in range(R):
        hp_ref[r, :, :] = jnp.dot(zs, w_ref[r], preferred_element_type=jnp.float32)


def _tck_mid(p, dis, b, w):
    return pl.pallas_call(
        _tck_mid_body,
        grid=(N // _BLK,),
        in_specs=[
            pl.BlockSpec((NC, _BLK, D), lambda i: (0, i, 0)),
            pl.BlockSpec((_BLK, 1), lambda i: (i, 0)),
            pl.BlockSpec((1, D), lambda i: (0, 0)),
            pl.BlockSpec((R, D, D), lambda i: (0, 0, 0)),
        ],
        out_specs=[
            pl.BlockSpec((_BLK, D), lambda i: (i, 0)),
            pl.BlockSpec((R, _BLK, D), lambda i: (0, i, 0)),
        ],
        out_shape=[
            jax.ShapeDtypeStruct((N, D), jnp.float32),
            jax.ShapeDtypeStruct((R, N, D), jnp.float32),
        ],
    )(p, dis, b, w)


def _tck_fin_body(p_ref, dis_ref, b_ref, x_ref, z1_ref, z2_ref, z_ref):
    v = (p_ref[0] + p_ref[1]) * dis_ref[...] + b_ref[...]
    z3 = jnp.where(v >= 0, v, 0.01 * v)
    z_ref[...] = (x_ref[...] + z1_ref[...] + z2_ref[...] + z3) * 0.25


def _tck_fin(p, dis, b, x, z1, z2):
    return pl.pallas_call(
        _tck_fin_body,
        grid=(N // _BLK,),
        in_specs=[
            pl.BlockSpec((NC, _BLK, D), lambda i: (0, i, 0)),
            pl.BlockSpec((_BLK, 1), lambda i: (i, 0)),
            pl.BlockSpec((1, D), lambda i: (0, 0)),
            pl.BlockSpec((_BLK, D), lambda i: (i, 0)),
            pl.BlockSpec((_BLK, D), lambda i: (i, 0)),
            pl.BlockSpec((_BLK, D), lambda i: (i, 0)),
        ],
        out_specs=pl.BlockSpec((_BLK, D), lambda i: (i, 0)),
        out_shape=jax.ShapeDtypeStruct((N, D), jnp.float32),
    )(p, dis, b, x, z1, z2)


def kernel(x, edge_index, edge_type, edge_attr, w1, b1, w2, b2, w3, b3):
    pad = E_PAD - E
    row2 = jnp.concatenate([edge_index[0], jnp.zeros((pad,), jnp.int32)]).reshape(NW, EPT)
    col2 = jnp.concatenate([edge_index[1], jnp.full((pad,), N, jnp.int32)]).reshape(NW, EPT)
    et2 = jnp.concatenate([edge_type, jnp.zeros((pad,), jnp.int32)]).reshape(NW, EPT)
    attr2 = jnp.concatenate([edge_attr, jnp.zeros((pad,), jnp.float32)]).reshape(NW, EPT)

    gidx2, degp = _sck0(col2, et2)
    gidx3 = gidx2.reshape(NW, EPB, CHUNK)
    row3 = row2.reshape(NW, EPB, CHUNK)

    dis = _tck0(degp)

    hp1 = _tck1(x, dis, w1)
    p1 = _sck_edge(hp1.reshape(R * N, D), gidx3, attr2, row3)
    z1, hp2 = _tck_mid(p1, dis, b1.reshape(1, D), w2)
    p2 = _sck_edge(hp2.reshape(R * N, D), gidx3, attr2, row3)
    z2, hp3 = _tck_mid(p2, dis, b2.reshape(1, D), w3)
    p3 = _sck_edge(hp3.reshape(R * N, D), gidx3, attr2, row3)
    z = _tck_fin(p3, dis, b3.reshape(1, D), x, z1, z2)
    return z


# speedup vs baseline: 3.9050x; 1.4660x over previous
import functools

import jax
import jax.numpy as jnp
from jax import lax
from jax.experimental import pallas as pl
from jax.experimental.pallas import tpu as pltpu
from jax.experimental.pallas import tpu_sc as plsc

N = 10000
E = 320000
D = 128
R = 4

NC = 2
NS = 16
NW = NC * NS

CHUNK = 80
EPB = 126
EPT = EPB * CHUNK
E_PAD = NW * EPT
NPAD = 10240
RPT = N // NS

_BLK = 1000


def _mesh():
    return plsc.VectorSubcoreMesh(
        core_axis_name="c", subcore_axis_name="s", num_cores=NC, num_subcores=NS
    )


_SC_PARAMS = dict(
    compiler_params=pltpu.CompilerParams(
        needs_layout_passes=False, use_tc_tiling_on_sc=False
    ),
)


def _sck0_body(col_h, et_h, gidx_h, degp_h, colv, etv, gv, degv):
    c = lax.axis_index("c")
    s = lax.axis_index("s")
    w = c * NS + s
    pltpu.sync_copy(col_h.at[w], colv)
    pltpu.sync_copy(et_h.at[w], etv)

    zero16 = jnp.zeros((16,), jnp.float32)

    def _z(i, carry):
        degv[pl.ds(i * 16, 16)] = zero16
        return carry

    lax.fori_loop(0, NPAD // 16, _z, 0)

    ones16 = jnp.ones((16,), jnp.float32)

    def _e(k, carry):
        cc = colv[pl.ds(k * 16, 16)]
        tt = etv[pl.ds(k * 16, 16)]
        gv[pl.ds(k * 16, 16)] = tt * N + cc
        plsc.addupdate_scatter(degv, [cc], ones16)
        return carry

    lax.fori_loop(0, EPT // 16, _e, 0)

    pltpu.sync_copy(gv, gidx_h.at[w])
    pltpu.sync_copy(degv, degp_h.at[w])


def _sck0(col2, et2):
    return pl.kernel(
        _sck0_body,
        out_type=(
            jax.ShapeDtypeStruct((NW, EPT), jnp.int32),
            jax.ShapeDtypeStruct((NW, NPAD), jnp.float32),
        ),
        mesh=_mesh(),
        scratch_types=[
            pltpu.VMEM((EPT,), jnp.int32),
            pltpu.VMEM((EPT,), jnp.int32),
            pltpu.VMEM((EPT,), jnp.int32),
            pltpu.VMEM((NPAD,), jnp.float32),
        ],
        **_SC_PARAMS,
    )(col2, et2)


def _sck_edge_body(hp_h, gidx_h, attr_h, row_h, p_h, gidxv, attrv, rowv,
                   gbufa, gbufb, acc, sema, semb):
    c = lax.axis_index("c")
    s = lax.axis_index("s")
    w = c * NS + s
    gb = [gbufa, gbufb]
    gsem = [sema, semb]
    pltpu.sync_copy(gidx_h.at[w], gidxv)
    pltpu.sync_copy(attr_h.at[w], attrv)
    pltpu.sync_copy(row_h.at[w], rowv)

    zero16 = jnp.zeros((16,), jnp.float32)

    def _zrow(i, carry):
        for q in range(D // 16):
            gbufa[i, pl.ds(q * 16, 16)] = zero16
        return carry

    lax.fori_loop(0, CHUNK, _zrow, 0)

    base = s * RPT
    for k in range(RPT // CHUNK):
        pltpu.sync_copy(gbufa, acc.at[pl.ds(base + k * CHUNK, CHUNK)])
    rem = RPT % CHUNK
    if rem:
        pltpu.sync_copy(gbufa.at[pl.ds(0, rem)],
                        acc.at[pl.ds(base + RPT - rem, rem)])

    pltpu.async_copy(hp_h.at[gidxv.at[0]], gbufa, gsem[0])
    plsc.subcore_barrier()

    def _iter(j, b):
        pltpu.make_async_copy(hp_h.at[gidxv.at[j]], gb[b], gsem[b]).wait()

        @pl.when(j + 1 < EPB)
        def _next():
            pltpu.async_copy(hp_h.at[gidxv.at[j + 1]], gb[1 - b], gsem[1 - b])

        def _srow(i, carry2):
            a = plsc.load_gather(attrv, [jnp.full((16,), j * CHUNK + i, jnp.int32)])
            for q in range(D // 16):
                gb[b][i, pl.ds(q * 16, 16)] = gb[b][i, pl.ds(q * 16, 16)] * a
            return carry2

        lax.fori_loop(0, CHUNK, _srow, 0, unroll=4)
        pltpu.sync_copy(gb[b], acc.at[rowv.at[j]], add=True)

    def _outer(t, carry):
        for b in range(2):
            _iter(t * 2 + b, b)
        return carry

    lax.fori_loop(0, EPB // 2, _outer, 0)

    plsc.subcore_barrier()
    pltpu.sync_copy(acc.at[pl.ds(base, RPT)], p_h.at[c, pl.ds(base, RPT)])


def _sck_edge(hp_flat, gidx3, attr2, row3):
    return pl.kernel(
        _sck_edge_body,
        out_type=jax.ShapeDtypeStruct((NC, N, D), jnp.float32),
        mesh=_mesh(),
        scratch_types=[
            pltpu.VMEM((EPB, CHUNK), jnp.int32),
            pltpu.VMEM((EPT,), jnp.float32),
            pltpu.VMEM((EPB, CHUNK), jnp.int32),
            pltpu.VMEM((CHUNK, D), jnp.float32),
            pltpu.VMEM((CHUNK, D), jnp.float32),
            pltpu.VMEM_SHARED((N, D), jnp.float32),
            pltpu.SemaphoreType.DMA,
            pltpu.SemaphoreType.DMA,
        ],
        **_SC_PARAMS,
    )(hp_flat, gidx3, attr2, row3)


def _tck0_body(degp_ref, dis_ref):
    deg = jnp.sum(degp_ref[...], axis=0)
    dis = jnp.where(deg > 0, lax.rsqrt(deg), 0.0)
    dis_ref[...] = dis[:, None]


def _tck0(degp):
    return pl.pallas_call(
        _tck0_body,
        out_shape=jax.ShapeDtypeStruct((NPAD, 1), jnp.float32),
    )(degp)


def _tck1_body(x_ref, dis_ref, w_ref, hp_ref):
    xs = x_ref[...] * dis_ref[...]
    for r in range(R):
        hp_ref[r, :, :] = jnp.dot(xs, w_ref[r], preferred_element_type=jnp.float32)


def _tck1(x, dis, w):
    return pl.pallas_call(
        _tck1_body,
        grid=(N // _BLK,),
        in_specs=[
            pl.BlockSpec((_BLK, D), lambda i: (i, 0)),
            pl.BlockSpec((_BLK, 1), lambda i: (i, 0)),
            pl.BlockSpec((R, D, D), lambda i: (0, 0, 0)),
        ],
        out_specs=pl.BlockSpec((R, _BLK, D), lambda i: (0, i, 0)),
        out_shape=jax.ShapeDtypeStruct((R, N, D), jnp.float32),
    )(x, dis, w)


def _tck_mid_body(p_ref, dis_ref, b_ref, w_ref, z_ref, hp_ref):
    dis = dis_ref[...]
    v = (p_ref[0] + p_ref[1]) * dis + b_ref[...]
    z = jnp.where(v >= 0, v, 0.01 * v)
    z_ref[...] = z
    zs = z * dis
    for r in range(R):
        hp_ref[r, :, :] = jnp.dot(zs, w_ref[r], preferred_element_type=jnp.float32)


def _tck_mid(p, dis, b, w):
    return pl.pallas_call(
        _tck_mid_body,
        grid=(N // _BLK,),
        in_specs=[
            pl.BlockSpec((NC, _BLK, D), lambda i: (0, i, 0)),
            pl.BlockSpec((_BLK, 1), lambda i: (i, 0)),
            pl.BlockSpec((1, D), lambda i: (0, 0)),
            pl.BlockSpec((R, D, D), lambda i: (0, 0, 0)),
        ],
        out_specs=[
            pl.BlockSpec((_BLK, D), lambda i: (i, 0)),
            pl.BlockSpec((R, _BLK, D), lambda i: (0, i, 0)),
        ],
        out_shape=[
            jax.ShapeDtypeStruct((N, D), jnp.float32),
            jax.ShapeDtypeStruct((R, N, D), jnp.float32),
        ],
    )(p, dis, b, w)


def _tck_fin_body(p_ref, dis_ref, b_ref, x_ref, z1_ref, z2_ref, z_ref):
    v = (p_ref[0] + p_ref[1]) * dis_ref[...] + b_ref[...]
    z3 = jnp.where(v >= 0, v, 0.01 * v)
    z_ref[...] = (x_ref[...] + z1_ref[...] + z2_ref[...] + z3) * 0.25


def _tck_fin(p, dis, b, x, z1, z2):
    return pl.pallas_call(
        _tck_fin_body,
        grid=(N // _BLK,),
        in_specs=[
            pl.BlockSpec((NC, _BLK, D), lambda i: (0, i, 0)),
            pl.BlockSpec((_BLK, 1), lambda i: (i, 0)),
            pl.BlockSpec((1, D), lambda i: (0, 0)),
            pl.BlockSpec((_BLK, D), lambda i: (i, 0)),
            pl.BlockSpec((_BLK, D), lambda i: (i, 0)),
            pl.BlockSpec((_BLK, D), lambda i: (i, 0)),
        ],
        out_specs=pl.BlockSpec((_BLK, D), lambda i: (i, 0)),
        out_shape=jax.ShapeDtypeStruct((N, D), jnp.float32),
    )(p, dis, b, x, z1, z2)


def kernel(x, edge_index, edge_type, edge_attr, w1, b1, w2, b2, w3, b3):
    pad = E_PAD - E
    row2 = jnp.concatenate([edge_index[0], jnp.zeros((pad,), jnp.int32)]).reshape(NW, EPT)
    col2 = jnp.concatenate([edge_index[1], jnp.full((pad,), N, jnp.int32)]).reshape(NW, EPT)
    et2 = jnp.concatenate([edge_type, jnp.zeros((pad,), jnp.int32)]).reshape(NW, EPT)
    attr2 = jnp.concatenate([edge_attr, jnp.zeros((pad,), jnp.float32)]).reshape(NW, EPT)

    gidx2, degp = _sck0(col2, et2)
    gidx3 = gidx2.reshape(NW, EPB, CHUNK)
    row3 = row2.reshape(NW, EPB, CHUNK)

    dis = _tck0(degp)

    hp1 = _tck1(x, dis, w1)
    p1 = _sck_edge(hp1.reshape(R * N, D), gidx3, attr2, row3)
    z1, hp2 = _tck_mid(p1, dis, b1.reshape(1, D), w2)
    p2 = _sck_edge(hp2.reshape(R * N, D), gidx3, attr2, row3)
    z2, hp3 = _tck_mid(p2, dis, b2.reshape(1, D), w3)
    p3 = _sck_edge(hp3.reshape(R * N, D), gidx3, attr2, row3)
    z = _tck_fin(p3, dis, b3.reshape(1, D), x, z1, z2)
    return z


# speedup vs baseline: 4.9165x; 1.2590x over previous
import functools

import jax
import jax.numpy as jnp
from jax import lax
from jax.experimental import pallas as pl
from jax.experimental.pallas import tpu as pltpu
from jax.experimental.pallas import tpu_sc as plsc

N = 10000
E = 320000
D = 128
R = 4

NC = 2
NS = 16
NW = NC * NS

CHUNK = 80
EPB_A = 148
EPB_B = 102
EPT0 = E // NW
NPAD = 10240
RPT = N // NS

_BLK = 1000


def _mesh():
    return plsc.VectorSubcoreMesh(
        core_axis_name="c", subcore_axis_name="s", num_cores=NC, num_subcores=NS
    )


_SC_PARAMS = dict(
    compiler_params=pltpu.CompilerParams(
        needs_layout_passes=False, use_tc_tiling_on_sc=False
    ),
)


def _sck0_body(col_h, et_h, row_h, gidx_h, degp_h, colv, etv, rowv, gv, degv):
    c = lax.axis_index("c")
    s = lax.axis_index("s")
    w = c * NS + s
    pltpu.sync_copy(col_h.at[w], colv)
    pltpu.sync_copy(et_h.at[w], etv)
    pltpu.sync_copy(row_h.at[w], rowv)

    zero16 = jnp.zeros((16,), jnp.float32)

    def _z(i, carry):
        degv[pl.ds(i * 16, 16)] = zero16
        return carry

    lax.fori_loop(0, NPAD // 16, _z, 0)

    ones16 = jnp.ones((16,), jnp.float32)

    def _e(k, carry):
        cc = colv[pl.ds(k * 16, 16)]
        tt = etv[pl.ds(k * 16, 16)]
        rr = rowv[pl.ds(k * 16, 16)]
        gv[pl.ds(k * 16, 16)] = (tt * N + cc) | (rr << 16)
        plsc.addupdate_scatter(degv, [cc], ones16)
        return carry

    lax.fori_loop(0, EPT0 // 16, _e, 0)

    pltpu.sync_copy(gv, gidx_h.at[w])
    pltpu.sync_copy(degv, degp_h.at[w])


def _sck0(col2, et2, row2):
    return pl.kernel(
        _sck0_body,
        out_type=(
            jax.ShapeDtypeStruct((NW, EPT0), jnp.int32),
            jax.ShapeDtypeStruct((NW, NPAD), jnp.float32),
        ),
        mesh=_mesh(),
        scratch_types=[
            pltpu.VMEM((EPT0,), jnp.int32),
            pltpu.VMEM((EPT0,), jnp.int32),
            pltpu.VMEM((EPT0,), jnp.int32),
            pltpu.VMEM((EPT0,), jnp.int32),
            pltpu.VMEM((NPAD,), jnp.float32),
        ],
        **_SC_PARAMS,
    )(col2, et2, row2)


def _sck_edge_body(hp_h, pk_h, attr_h, p_h,
                   pkv, attrv, gixa, gixb, rwa, rwb,
                   gbufa, gbufb, acc, sema, semb):
    c = lax.axis_index("c")
    s = lax.axis_index("s")
    gb = [gbufa, gbufb]
    gix = [gixa, gixb]
    rw = [rwa, rwb]
    gsem = [sema, semb]

    @pl.when(c == 0)
    def _stage_a():
        eb = s * (EPB_A * CHUNK)
        pltpu.sync_copy(pk_h.at[pl.ds(eb, EPB_A * CHUNK)], pkv.at[pl.ds(0, EPB_A * CHUNK)])
        pltpu.sync_copy(attr_h.at[pl.ds(eb, EPB_A * CHUNK)], attrv.at[pl.ds(0, EPB_A * CHUNK)])

    @pl.when(c == 1)
    def _stage_b():
        eb = NS * (EPB_A * CHUNK) + s * (EPB_B * CHUNK)
        pltpu.sync_copy(pk_h.at[pl.ds(eb, EPB_B * CHUNK)], pkv.at[pl.ds(0, EPB_B * CHUNK)])
        pltpu.sync_copy(attr_h.at[pl.ds(eb, EPB_B * CHUNK)], attrv.at[pl.ds(0, EPB_B * CHUNK)])

    nmine = jnp.where(c == 0, EPB_A, EPB_B)

    zero16 = jnp.zeros((16,), jnp.float32)

    def _zrow(i, carry):
        for q in range(D // 16):
            gbufa[i, pl.ds(q * 16, 16)] = zero16
        return carry

    lax.fori_loop(0, CHUNK, _zrow, 0)

    base = s * RPT
    for k in range(RPT // CHUNK):
        pltpu.sync_copy(gbufa, acc.at[pl.ds(base + k * CHUNK, CHUNK)])
    rem = RPT % CHUNK
    if rem:
        pltpu.sync_copy(gbufa.at[pl.ds(0, rem)],
                        acc.at[pl.ds(base + RPT - rem, rem)])

    mask16 = jnp.full((16,), 0xFFFF, jnp.int32)

    def _unpack(j, slot):
        def _u(k, carry):
            pk = pkv[pl.ds(j * CHUNK + k * 16, 16)]
            gix[slot][pl.ds(k * 16, 16)] = pk & mask16
            rw[slot][pl.ds(k * 16, 16)] = lax.shift_right_logical(pk, 16)
            return carry
        lax.fori_loop(0, CHUNK // 16, _u, 0)

    _unpack(0, 0)
    pltpu.async_copy(hp_h.at[gixa], gbufa, gsem[0])
    plsc.subcore_barrier()

    def _iter(j, b):
        pltpu.make_async_copy(hp_h.at[gix[b]], gb[b], gsem[b]).wait()

        @pl.when(j + 1 < nmine)
        def _next():
            _unpack(j + 1, 1 - b)
            pltpu.async_copy(hp_h.at[gix[1 - b]], gb[1 - b], gsem[1 - b])

        def _srow(i, carry2):
            a = plsc.load_gather(attrv, [jnp.full((16,), j * CHUNK + i, jnp.int32)])
            for q in range(D // 16):
                gb[b][i, pl.ds(q * 16, 16)] = gb[b][i, pl.ds(q * 16, 16)] * a
            return carry2

        lax.fori_loop(0, CHUNK, _srow, 0, unroll=4)
        pltpu.sync_copy(gb[b], acc.at[rw[b]], add=True)

    def _outer(t, carry):
        for b in range(2):
            _iter(t * 2 + b, b)
        return carry

    lax.fori_loop(0, nmine // 2, _outer, 0)

    plsc.subcore_barrier()
    pltpu.sync_copy(acc.at[pl.ds(base, RPT)], p_h.at[c, pl.ds(base, RPT)])


def _sck_edge(hp_flat, pk, attr):
    return pl.kernel(
        _sck_edge_body,
        out_type=jax.ShapeDtypeStruct((NC, N, D), jnp.float32),
        mesh=_mesh(),
        scratch_types=[
            pltpu.VMEM((EPB_A * CHUNK,), jnp.int32),
            pltpu.VMEM((EPB_A * CHUNK,), jnp.float32),
            pltpu.VMEM((CHUNK,), jnp.int32),
            pltpu.VMEM((CHUNK,), jnp.int32),
            pltpu.VMEM((CHUNK,), jnp.int32),
            pltpu.VMEM((CHUNK,), jnp.int32),
            pltpu.VMEM((CHUNK, D), jnp.float32),
            pltpu.VMEM((CHUNK, D), jnp.float32),
            pltpu.VMEM_SHARED((N, D), jnp.float32),
            pltpu.SemaphoreType.DMA,
            pltpu.SemaphoreType.DMA,
        ],
        **_SC_PARAMS,
    )(hp_flat, pk, attr)


def _tck0_body(degp_ref, dis_ref):
    deg = jnp.sum(degp_ref[...], axis=0)
    dis = jnp.where(deg > 0, lax.rsqrt(deg), 0.0)
    dis_ref[...] = dis[:, None]


def _tck0(degp):
    return pl.pallas_call(
        _tck0_body,
        out_shape=jax.ShapeDtypeStruct((NPAD, 1), jnp.float32),
    )(degp)


def _tck1_body(x_ref, dis_ref, w_ref, hp_ref):
    xs = x_ref[...] * dis_ref[...]
    for r in range(R):
        hp_ref[r, :, :] = jnp.dot(xs, w_ref[r], preferred_element_type=jnp.float32)


def _tck1(x, dis, w):
    return pl.pallas_call(
        _tck1_body,
        grid=(N // _BLK,),
        in_specs=[
            pl.BlockSpec((_BLK, D), lambda i: (i, 0)),
            pl.BlockSpec((_BLK, 1), lambda i: (i, 0)),
            pl.BlockSpec((R, D, D), lambda i: (0, 0, 0)),
        ],
        out_specs=pl.BlockSpec((R, _BLK, D), lambda i: (0, i, 0)),
        out_shape=jax.ShapeDtypeStruct((R, N, D), jnp.float32),
    )(x, dis, w)


def _tck_mid_body(p_ref, dis_ref, b_ref, w_ref, z_ref, hp_ref):
    dis = dis_ref[...]
    v = (p_ref[0] + p_ref[1]) * dis + b_ref[...]
    z = jnp.where(v >= 0, v, 0.01 * v)
    z_ref[...] = z
    zs = z * dis
    for r in range(R):
        hp_ref[r, :, :] = jnp.dot(zs, w_ref[r], preferred_element_type=jnp.float32)


def _tck_mid(p, dis, b, w):
    return pl.pallas_call(
        _tck_mid_body,
        grid=(N // _BLK,),
        in_specs=[
            pl.BlockSpec((NC, _BLK, D), lambda i: (0, i, 0)),
            pl.BlockSpec((_BLK, 1), lambda i: (i, 0)),
            pl.BlockSpec((1, D), lambda i: (0, 0)),
            pl.BlockSpec((R, D, D), lambda i: (0, 0, 0)),
        ],
        out_specs=[
            pl.BlockSpec((_BLK, D), lambda i: (i, 0)),
            pl.BlockSpec((R, _BLK, D), lambda i: (0, i, 0)),
        ],
        out_shape=[
            jax.ShapeDtypeStruct((N, D), jnp.float32),
            jax.ShapeDtypeStruct((R, N, D), jnp.float32),
        ],
    )(p, dis, b, w)


def _tck_fin_body(p_ref, dis_ref, b_ref, x_ref, z1_ref, z2_ref, z_ref):
    v = (p_ref[0] + p_ref[1]) * dis_ref[...] + b_ref[...]
    z3 = jnp.where(v >= 0, v, 0.01 * v)
    z_ref[...] = (x_ref[...] + z1_ref[...] + z2_ref[...] + z3) * 0.25


def _tck_fin(p, dis, b, x, z1, z2):
    return pl.pallas_call(
        _tck_fin_body,
        grid=(N // _BLK,),
        in_specs=[
            pl.BlockSpec((NC, _BLK, D), lambda i: (0, i, 0)),
            pl.BlockSpec((_BLK, 1), lambda i: (i, 0)),
            pl.BlockSpec((1, D), lambda i: (0, 0)),
            pl.BlockSpec((_BLK, D), lambda i: (i, 0)),
            pl.BlockSpec((_BLK, D), lambda i: (i, 0)),
            pl.BlockSpec((_BLK, D), lambda i: (i, 0)),
        ],
        out_specs=pl.BlockSpec((_BLK, D), lambda i: (i, 0)),
        out_shape=jax.ShapeDtypeStruct((N, D), jnp.float32),
    )(p, dis, b, x, z1, z2)


def kernel(x, edge_index, edge_type, edge_attr, w1, b1, w2, b2, w3, b3):
    row2 = edge_index[0].reshape(NW, EPT0)
    col2 = edge_index[1].reshape(NW, EPT0)
    et2 = edge_type.reshape(NW, EPT0)

    pk2, degp = _sck0(col2, et2, row2)
    pk = pk2.reshape(E)

    dis = _tck0(degp)

    hp1 = _tck1(x, dis, w1)
    p1 = _sck_edge(hp1.reshape(R * N, D), pk, edge_attr)
    z1, hp2 = _tck_mid(p1, dis, b1.reshape(1, D), w2)
    p2 = _sck_edge(hp2.reshape(R * N, D), pk, edge_attr)
    z2, hp3 = _tck_mid(p2, dis, b2.reshape(1, D), w3)
    p3 = _sck_edge(hp3.reshape(R * N, D), pk, edge_attr)
    z = _tck_fin(p3, dis, b3.reshape(1, D), x, z1, z2)
    return z


# speedup vs baseline: 5.2363x; 1.0651x over previous
import functools

import jax
import jax.numpy as jnp
from jax import lax
from jax.experimental import pallas as pl
from jax.experimental.pallas import tpu as pltpu
from jax.experimental.pallas import tpu_sc as plsc

N = 10000
E = 320000
D = 128
R = 4

NC = 2
NS = 16
NW = NC * NS

CHUNK = 80
EPB_A = 136
EPB_B = 114
EPT0 = E // NW
NPAD = 10240
RPT = N // NS

_BLK = 1000


def _mesh():
    return plsc.VectorSubcoreMesh(
        core_axis_name="c", subcore_axis_name="s", num_cores=NC, num_subcores=NS
    )


_SC_PARAMS = dict(
    compiler_params=pltpu.CompilerParams(
        needs_layout_passes=False, use_tc_tiling_on_sc=False
    ),
)


def _sck0_body(col_h, et_h, row_h, gidx_h, degp_h, colv, etv, rowv, gv, degv):
    c = lax.axis_index("c")
    s = lax.axis_index("s")
    w = c * NS + s
    pltpu.sync_copy(col_h.at[w], colv)
    pltpu.sync_copy(et_h.at[w], etv)
    pltpu.sync_copy(row_h.at[w], rowv)

    zero16 = jnp.zeros((16,), jnp.float32)

    def _z(i, carry):
        degv[pl.ds(i * 16, 16)] = zero16
        return carry

    lax.fori_loop(0, NPAD // 16, _z, 0)

    ones16 = jnp.ones((16,), jnp.float32)

    def _e(k, carry):
        cc = colv[pl.ds(k * 16, 16)]
        tt = etv[pl.ds(k * 16, 16)]
        rr = rowv[pl.ds(k * 16, 16)]
        gv[pl.ds(k * 16, 16)] = (tt * N + cc) | (rr << 16)
        plsc.addupdate_scatter(degv, [cc], ones16)
        return carry

    lax.fori_loop(0, EPT0 // 16, _e, 0)

    pltpu.sync_copy(gv, gidx_h.at[w])
    pltpu.sync_copy(degv, degp_h.at[w])


def _sck0(col2, et2, row2):
    return pl.kernel(
        _sck0_body,
        out_type=(
            jax.ShapeDtypeStruct((NW, EPT0), jnp.int32),
            jax.ShapeDtypeStruct((NW, NPAD), jnp.float32),
        ),
        mesh=_mesh(),
        scratch_types=[
            pltpu.VMEM((EPT0,), jnp.int32),
            pltpu.VMEM((EPT0,), jnp.int32),
            pltpu.VMEM((EPT0,), jnp.int32),
            pltpu.VMEM((EPT0,), jnp.int32),
            pltpu.VMEM((NPAD,), jnp.float32),
        ],
        **_SC_PARAMS,
    )(col2, et2, row2)


def _sck_edge_body(hp_h, pk_h, attr_h, p_h,
                   pkv, attrv, gixa, gixb, rwa, rwb,
                   gbufa, gbufb, acc, sema, semb):
    c = lax.axis_index("c")
    s = lax.axis_index("s")
    gb = [gbufa, gbufb]
    gix = [gixa, gixb]
    rw = [rwa, rwb]
    gsem = [sema, semb]

    @pl.when(c == 0)
    def _stage_a():
        eb = s * (EPB_A * CHUNK)
        pltpu.sync_copy(pk_h.at[pl.ds(eb, EPB_A * CHUNK)], pkv.at[pl.ds(0, EPB_A * CHUNK)])
        pltpu.sync_copy(attr_h.at[pl.ds(eb, EPB_A * CHUNK)], attrv.at[pl.ds(0, EPB_A * CHUNK)])

    @pl.when(c == 1)
    def _stage_b():
        eb = NS * (EPB_A * CHUNK) + s * (EPB_B * CHUNK)
        pltpu.sync_copy(pk_h.at[pl.ds(eb, EPB_B * CHUNK)], pkv.at[pl.ds(0, EPB_B * CHUNK)])
        pltpu.sync_copy(attr_h.at[pl.ds(eb, EPB_B * CHUNK)], attrv.at[pl.ds(0, EPB_B * CHUNK)])

    nmine = jnp.where(c == 0, EPB_A, EPB_B)

    zero16 = jnp.zeros((16,), jnp.float32)

    def _zrow(i, carry):
        for q in range(D // 16):
            gbufa[i, pl.ds(q * 16, 16)] = zero16
        return carry

    lax.fori_loop(0, CHUNK, _zrow, 0)

    base = s * RPT
    for k in range(RPT // CHUNK):
        pltpu.sync_copy(gbufa, acc.at[pl.ds(base + k * CHUNK, CHUNK)])
    rem = RPT % CHUNK
    if rem:
        pltpu.sync_copy(gbufa.at[pl.ds(0, rem)],
                        acc.at[pl.ds(base + RPT - rem, rem)])

    mask16 = jnp.full((16,), 0xFFFF, jnp.int32)

    def _unpack(j, slot):
        def _u(k, carry):
            pk = pkv[pl.ds(j * CHUNK + k * 16, 16)]
            gix[slot][pl.ds(k * 16, 16)] = pk & mask16
            rw[slot][pl.ds(k * 16, 16)] = lax.shift_right_logical(pk, 16)
            return carry
        lax.fori_loop(0, CHUNK // 16, _u, 0)

    _unpack(0, 0)
    pltpu.async_copy(hp_h.at[gixa], gbufa, gsem[0])
    plsc.subcore_barrier()

    def _iter(j, b):
        pltpu.make_async_copy(hp_h.at[gix[b]], gb[b], gsem[b]).wait()

        @pl.when(j + 1 < nmine)
        def _next():
            _unpack(j + 1, 1 - b)
            pltpu.async_copy(hp_h.at[gix[1 - b]], gb[1 - b], gsem[1 - b])

        def _srow(i, carry2):
            a = plsc.load_gather(attrv, [jnp.full((16,), j * CHUNK + i, jnp.int32)])
            for q in range(D // 16):
                gb[b][i, pl.ds(q * 16, 16)] = gb[b][i, pl.ds(q * 16, 16)] * a
            return carry2

        lax.fori_loop(0, CHUNK, _srow, 0, unroll=4)
        pltpu.sync_copy(gb[b], acc.at[rw[b]], add=True)

    def _outer(t, carry):
        for b in range(2):
            _iter(t * 2 + b, b)
        return carry

    lax.fori_loop(0, nmine // 2, _outer, 0)

    plsc.subcore_barrier()
    pltpu.sync_copy(acc.at[pl.ds(base, RPT)], p_h.at[c, pl.ds(base, RPT)])


def _sck_edge(hp_flat, pk, attr):
    return pl.kernel(
        _sck_edge_body,
        out_type=jax.ShapeDtypeStruct((NC, N, D), jnp.float32),
        mesh=_mesh(),
        scratch_types=[
            pltpu.VMEM((EPB_A * CHUNK,), jnp.int32),
            pltpu.VMEM((EPB_A * CHUNK,), jnp.float32),
            pltpu.VMEM((CHUNK,), jnp.int32),
            pltpu.VMEM((CHUNK,), jnp.int32),
            pltpu.VMEM((CHUNK,), jnp.int32),
            pltpu.VMEM((CHUNK,), jnp.int32),
            pltpu.VMEM((CHUNK, D), jnp.float32),
            pltpu.VMEM((CHUNK, D), jnp.float32),
            pltpu.VMEM_SHARED((N, D), jnp.float32),
            pltpu.SemaphoreType.DMA,
            pltpu.SemaphoreType.DMA,
        ],
        **_SC_PARAMS,
    )(hp_flat, pk, attr)


def _tck0_body(degp_ref, dis_ref):
    deg = jnp.sum(degp_ref[...], axis=0)
    dis = jnp.where(deg > 0, lax.rsqrt(deg), 0.0)
    dis_ref[...] = dis[:, None]


def _tck0(degp):
    return pl.pallas_call(
        _tck0_body,
        out_shape=jax.ShapeDtypeStruct((NPAD, 1), jnp.float32),
    )(degp)


def _tck1_body(x_ref, dis_ref, w_ref, hp_ref):
    xs = x_ref[...] * dis_ref[...]
    for r in range(R):
        hp_ref[r, :, :] = jnp.dot(xs, w_ref[r], preferred_element_type=jnp.float32)


def _tck1(x, dis, w):
    return pl.pallas_call(
        _tck1_body,
        grid=(N // _BLK,),
        in_specs=[
            pl.BlockSpec((_BLK, D), lambda i: (i, 0)),
            pl.BlockSpec((_BLK, 1), lambda i: (i, 0)),
            pl.BlockSpec((R, D, D), lambda i: (0, 0, 0)),
        ],
        out_specs=pl.BlockSpec((R, _BLK, D), lambda i: (0, i, 0)),
        out_shape=jax.ShapeDtypeStruct((R, N, D), jnp.float32),
    )(x, dis, w)


def _tck_mid_body(p_ref, dis_ref, b_ref, w_ref, z_ref, hp_ref):
    dis = dis_ref[...]
    v = (p_ref[0] + p_ref[1]) * dis + b_ref[...]
    z = jnp.where(v >= 0, v, 0.01 * v)
    z_ref[...] = z
    zs = z * dis
    for r in range(R):
        hp_ref[r, :, :] = jnp.dot(zs, w_ref[r], preferred_element_type=jnp.float32)


def _tck_mid(p, dis, b, w):
    return pl.pallas_call(
        _tck_mid_body,
        grid=(N // _BLK,),
        in_specs=[
            pl.BlockSpec((NC, _BLK, D), lambda i: (0, i, 0)),
            pl.BlockSpec((_BLK, 1), lambda i: (i, 0)),
            pl.BlockSpec((1, D), lambda i: (0, 0)),
            pl.BlockSpec((R, D, D), lambda i: (0, 0, 0)),
        ],
        out_specs=[
            pl.BlockSpec((_BLK, D), lambda i: (i, 0)),
            pl.BlockSpec((R, _BLK, D), lambda i: (0, i, 0)),
        ],
        out_shape=[
            jax.ShapeDtypeStruct((N, D), jnp.float32),
            jax.ShapeDtypeStruct((R, N, D), jnp.float32),
        ],
    )(p, dis, b, w)


def _tck_fin_body(p_ref, dis_ref, b_ref, x_ref, z1_ref, z2_ref, z_ref):
    v = (p_ref[0] + p_ref[1]) * dis_ref[...] + b_ref[...]
    z3 = jnp.where(v >= 0, v, 0.01 * v)
    z_ref[...] = (x_ref[...] + z1_ref[...] + z2_ref[...] + z3) * 0.25


def _tck_fin(p, dis, b, x, z1, z2):
    return pl.pallas_call(
        _tck_fin_body,
        grid=(N // _BLK,),
        in_specs=[
            pl.BlockSpec((NC, _BLK, D), lambda i: (0, i, 0)),
            pl.BlockSpec((_BLK, 1), lambda i: (i, 0)),
            pl.BlockSpec((1, D), lambda i: (0, 0)),
            pl.BlockSpec((_BLK, D), lambda i: (i, 0)),
            pl.BlockSpec((_BLK, D), lambda i: (i, 0)),
            pl.BlockSpec((_BLK, D), lambda i: (i, 0)),
        ],
        out_specs=pl.BlockSpec((_BLK, D), lambda i: (i, 0)),
        out_shape=jax.ShapeDtypeStruct((N, D), jnp.float32),
    )(p, dis, b, x, z1, z2)


def kernel(x, edge_index, edge_type, edge_attr, w1, b1, w2, b2, w3, b3):
    row2 = edge_index[0].reshape(NW, EPT0)
    col2 = edge_index[1].reshape(NW, EPT0)
    et2 = edge_type.reshape(NW, EPT0)

    pk2, degp = _sck0(col2, et2, row2)
    pk = pk2.reshape(E)

    dis = _tck0(degp)

    hp1 = _tck1(x, dis, w1)
    p1 = _sck_edge(hp1.reshape(R * N, D), pk, edge_attr)
    z1, hp2 = _tck_mid(p1, dis, b1.reshape(1, D), w2)
    p2 = _sck_edge(hp2.reshape(R * N, D), pk, edge_attr)
    z2, hp3 = _tck_mid(p2, dis, b2.reshape(1, D), w3)
    p3 = _sck_edge(hp3.reshape(R * N, D), pk, edge_attr)
    z = _tck_fin(p3, dis, b3.reshape(1, D), x, z1, z2)
    return z


# speedup vs baseline: 5.4024x; 1.0317x over previous
import functools

import jax
import jax.numpy as jnp
from jax import lax
from jax.experimental import pallas as pl
from jax.experimental.pallas import tpu as pltpu
from jax.experimental.pallas import tpu_sc as plsc

N = 10000
E = 320000
D = 128
R = 4

NC = 2
NS = 16
NW = NC * NS

CHUNK = 80
EPB_A = 130
EPB_B = 120
EPT0 = E // NW
NPAD = 10240
RPT = N // NS

_BLK = 1000


def _mesh():
    return plsc.VectorSubcoreMesh(
        core_axis_name="c", subcore_axis_name="s", num_cores=NC, num_subcores=NS
    )


_SC_PARAMS = dict(
    compiler_params=pltpu.CompilerParams(
        needs_layout_passes=False, use_tc_tiling_on_sc=False
    ),
)


def _sck0_body(col_h, et_h, row_h, gidx_h, degp_h, colv, etv, rowv, gv, degv):
    c = lax.axis_index("c")
    s = lax.axis_index("s")
    w = c * NS + s
    pltpu.sync_copy(col_h.at[w], colv)
    pltpu.sync_copy(et_h.at[w], etv)
    pltpu.sync_copy(row_h.at[w], rowv)

    zero16 = jnp.zeros((16,), jnp.float32)

    def _z(i, carry):
        degv[pl.ds(i * 16, 16)] = zero16
        return carry

    lax.fori_loop(0, NPAD // 16, _z, 0)

    ones16 = jnp.ones((16,), jnp.float32)

    def _e(k, carry):
        cc = colv[pl.ds(k * 16, 16)]
        tt = etv[pl.ds(k * 16, 16)]
        rr = rowv[pl.ds(k * 16, 16)]
        gv[pl.ds(k * 16, 16)] = (tt * N + cc) | (rr << 16)
        plsc.addupdate_scatter(degv, [cc], ones16)
        return carry

    lax.fori_loop(0, EPT0 // 16, _e, 0)

    pltpu.sync_copy(gv, gidx_h.at[w])
    pltpu.sync_copy(degv, degp_h.at[w])


def _sck0(col2, et2, row2):
    return pl.kernel(
        _sck0_body,
        out_type=(
            jax.ShapeDtypeStruct((NW, EPT0), jnp.int32),
            jax.ShapeDtypeStruct((NW, NPAD), jnp.float32),
        ),
        mesh=_mesh(),
        scratch_types=[
            pltpu.VMEM((EPT0,), jnp.int32),
            pltpu.VMEM((EPT0,), jnp.int32),
            pltpu.VMEM((EPT0,), jnp.int32),
            pltpu.VMEM((EPT0,), jnp.int32),
            pltpu.VMEM((NPAD,), jnp.float32),
        ],
        **_SC_PARAMS,
    )(col2, et2, row2)


def _sck_edge_body(hp_h, pk_h, attr_h, p_h,
                   pkv, attrv, gixa, gixb, rwa, rwb,
                   gbufa, gbufb, acc, sema, semb):
    c = lax.axis_index("c")
    s = lax.axis_index("s")
    gb = [gbufa, gbufb]
    gix = [gixa, gixb]
    rw = [rwa, rwb]
    gsem = [sema, semb]

    @pl.when(c == 0)
    def _stage_a():
        eb = s * (EPB_A * CHUNK)
        pltpu.sync_copy(pk_h.at[pl.ds(eb, EPB_A * CHUNK)], pkv.at[pl.ds(0, EPB_A * CHUNK)])
        pltpu.sync_copy(attr_h.at[pl.ds(eb, EPB_A * CHUNK)], attrv.at[pl.ds(0, EPB_A * CHUNK)])

    @pl.when(c == 1)
    def _stage_b():
        eb = NS * (EPB_A * CHUNK) + s * (EPB_B * CHUNK)
        pltpu.sync_copy(pk_h.at[pl.ds(eb, EPB_B * CHUNK)], pkv.at[pl.ds(0, EPB_B * CHUNK)])
        pltpu.sync_copy(attr_h.at[pl.ds(eb, EPB_B * CHUNK)], attrv.at[pl.ds(0, EPB_B * CHUNK)])

    nmine = jnp.where(c == 0, EPB_A, EPB_B)

    zero16 = jnp.zeros((16,), jnp.float32)

    def _zrow(i, carry):
        for q in range(D // 16):
            gbufa[i, pl.ds(q * 16, 16)] = zero16
        return carry

    lax.fori_loop(0, CHUNK, _zrow, 0)

    base = s * RPT
    for k in range(RPT // CHUNK):
        pltpu.sync_copy(gbufa, acc.at[pl.ds(base + k * CHUNK, CHUNK)])
    rem = RPT % CHUNK
    if rem:
        pltpu.sync_copy(gbufa.at[pl.ds(0, rem)],
                        acc.at[pl.ds(base + RPT - rem, rem)])

    mask16 = jnp.full((16,), 0xFFFF, jnp.int32)

    def _unpack(j, slot):
        def _u(k, carry):
            pk = pkv[pl.ds(j * CHUNK + k * 16, 16)]
            gix[slot][pl.ds(k * 16, 16)] = pk & mask16
            rw[slot][pl.ds(k * 16, 16)] = lax.shift_right_logical(pk, 16)
            return carry
        lax.fori_loop(0, CHUNK // 16, _u, 0)

    _unpack(0, 0)
    pltpu.async_copy(hp_h.at[gixa], gbufa, gsem[0])
    plsc.subcore_barrier()

    def _iter(j, b):
        pltpu.make_async_copy(hp_h.at[gix[b]], gb[b], gsem[b]).wait()

        @pl.when(j + 1 < nmine)
        def _next():
            _unpack(j + 1, 1 - b)
            pltpu.async_copy(hp_h.at[gix[1 - b]], gb[1 - b], gsem[1 - b])

        def _srow(i, carry2):
            a = plsc.load_gather(attrv, [jnp.full((16,), j * CHUNK + i, jnp.int32)])
            for q in range(D // 16):
                gb[b][i, pl.ds(q * 16, 16)] = gb[b][i, pl.ds(q * 16, 16)] * a
            return carry2

        lax.fori_loop(0, CHUNK, _srow, 0, unroll=4)
        pltpu.sync_copy(gb[b], acc.at[rw[b]], add=True)

    def _outer(t, carry):
        for b in range(2):
            _iter(t * 2 + b, b)
        return carry

    lax.fori_loop(0, nmine // 2, _outer, 0)

    plsc.subcore_barrier()
    pltpu.sync_copy(acc.at[pl.ds(base, RPT)], p_h.at[c, pl.ds(base, RPT)])


def _sck_edge(hp_flat, pk, attr):
    return pl.kernel(
        _sck_edge_body,
        out_type=jax.ShapeDtypeStruct((NC, N, D), jnp.float32),
        mesh=_mesh(),
        scratch_types=[
            pltpu.VMEM((EPB_A * CHUNK,), jnp.int32),
            pltpu.VMEM((EPB_A * CHUNK,), jnp.float32),
            pltpu.VMEM((CHUNK,), jnp.int32),
            pltpu.VMEM((CHUNK,), jnp.int32),
            pltpu.VMEM((CHUNK,), jnp.int32),
            pltpu.VMEM((CHUNK,), jnp.int32),
            pltpu.VMEM((CHUNK, D), jnp.float32),
            pltpu.VMEM((CHUNK, D), jnp.float32),
            pltpu.VMEM_SHARED((N, D), jnp.float32),
            pltpu.SemaphoreType.DMA,
            pltpu.SemaphoreType.DMA,
        ],
        **_SC_PARAMS,
    )(hp_flat, pk, attr)


def _tck0_body(degp_ref, dis_ref):
    deg = jnp.sum(degp_ref[...], axis=0)
    dis = jnp.where(deg > 0, lax.rsqrt(deg), 0.0)
    dis_ref[...] = dis[:, None]


def _tck0(degp):
    return pl.pallas_call(
        _tck0_body,
        out_shape=jax.ShapeDtypeStruct((NPAD, 1), jnp.float32),
    )(degp)


def _tck1_body(x_ref, dis_ref, w_ref, hp_ref):
    xs = x_ref[...] * dis_ref[...]
    for r in range(R):
        hp_ref[r, :, :] = jnp.dot(xs, w_ref[r], preferred_element_type=jnp.float32)


def _tck1(x, dis, w):
    return pl.pallas_call(
        _tck1_body,
        grid=(N // _BLK,),
        in_specs=[
            pl.BlockSpec((_BLK, D), lambda i: (i, 0)),
            pl.BlockSpec((_BLK, 1), lambda i: (i, 0)),
            pl.BlockSpec((R, D, D), lambda i: (0, 0, 0)),
        ],
        out_specs=pl.BlockSpec((R, _BLK, D), lambda i: (0, i, 0)),
        out_shape=jax.ShapeDtypeStruct((R, N, D), jnp.float32),
    )(x, dis, w)


def _tck_mid_body(p_ref, dis_ref, b_ref, w_ref, z_ref, hp_ref):
    dis = dis_ref[...]
    v = (p_ref[0] + p_ref[1]) * dis + b_ref[...]
    z = jnp.where(v >= 0, v, 0.01 * v)
    z_ref[...] = z
    zs = z * dis
    for r in range(R):
        hp_ref[r, :, :] = jnp.dot(zs, w_ref[r], preferred_element_type=jnp.float32)


def _tck_mid(p, dis, b, w):
    return pl.pallas_call(
        _tck_mid_body,
        grid=(N // _BLK,),
        in_specs=[
            pl.BlockSpec((NC, _BLK, D), lambda i: (0, i, 0)),
            pl.BlockSpec((_BLK, 1), lambda i: (i, 0)),
            pl.BlockSpec((1, D), lambda i: (0, 0)),
            pl.BlockSpec((R, D, D), lambda i: (0, 0, 0)),
        ],
        out_specs=[
            pl.BlockSpec((_BLK, D), lambda i: (i, 0)),
            pl.BlockSpec((R, _BLK, D), lambda i: (0, i, 0)),
        ],
        out_shape=[
            jax.ShapeDtypeStruct((N, D), jnp.float32),
            jax.ShapeDtypeStruct((R, N, D), jnp.float32),
        ],
    )(p, dis, b, w)


def _tck_fin_body(p_ref, dis_ref, b_ref, x_ref, z1_ref, z2_ref, z_ref):
    v = (p_ref[0] + p_ref[1]) * dis_ref[...] + b_ref[...]
    z3 = jnp.where(v >= 0, v, 0.01 * v)
    z_ref[...] = (x_ref[...] + z1_ref[...] + z2_ref[...] + z3) * 0.25


def _tck_fin(p, dis, b, x, z1, z2):
    return pl.pallas_call(
        _tck_fin_body,
        grid=(N // _BLK,),
        in_specs=[
            pl.BlockSpec((NC, _BLK, D), lambda i: (0, i, 0)),
            pl.BlockSpec((_BLK, 1), lambda i: (i, 0)),
            pl.BlockSpec((1, D), lambda i: (0, 0)),
            pl.BlockSpec((_BLK, D), lambda i: (i, 0)),
            pl.BlockSpec((_BLK, D), lambda i: (i, 0)),
            pl.BlockSpec((_BLK, D), lambda i: (i, 0)),
        ],
        out_specs=pl.BlockSpec((_BLK, D), lambda i: (i, 0)),
        out_shape=jax.ShapeDtypeStruct((N, D), jnp.float32),
    )(p, dis, b, x, z1, z2)


def kernel(x, edge_index, edge_type, edge_attr, w1, b1, w2, b2, w3, b3):
    row2 = edge_index[0].reshape(NW, EPT0)
    col2 = edge_index[1].reshape(NW, EPT0)
    et2 = edge_type.reshape(NW, EPT0)

    pk2, degp = _sck0(col2, et2, row2)
    pk = pk2.reshape(E)

    dis = _tck0(degp)

    hp1 = _tck1(x, dis, w1)
    p1 = _sck_edge(hp1.reshape(R * N, D), pk, edge_attr)
    z1, hp2 = _tck_mid(p1, dis, b1.reshape(1, D), w2)
    p2 = _sck_edge(hp2.reshape(R * N, D), pk, edge_attr)
    z2, hp3 = _tck_mid(p2, dis, b2.reshape(1, D), w3)
    p3 = _sck_edge(hp3.reshape(R * N, D), pk, edge_attr)
    z = _tck_fin(p3, dis, b3.reshape(1, D), x, z1, z2)
    return z


# speedup vs baseline: 5.5322x; 1.0240x over previous
import functools

import jax
import jax.numpy as jnp
from jax import lax
from jax.experimental import pallas as pl
from jax.experimental.pallas import tpu as pltpu
from jax.experimental.pallas import tpu_sc as plsc

N = 10000
E = 320000
D = 128
R = 4

NC = 2
NS = 16
NW = NC * NS

CHUNK = 80
EPB_A = 126
EPB_B = 124
EPT0 = E // NW
NPAD = 10240
RPT = N // NS

_BLK = 1000


def _mesh():
    return plsc.VectorSubcoreMesh(
        core_axis_name="c", subcore_axis_name="s", num_cores=NC, num_subcores=NS
    )


_SC_PARAMS = dict(
    compiler_params=pltpu.CompilerParams(
        needs_layout_passes=False, use_tc_tiling_on_sc=False
    ),
)


def _sck0_body(col_h, et_h, row_h, gidx_h, degp_h, colv, etv, rowv, gv, degv):
    c = lax.axis_index("c")
    s = lax.axis_index("s")
    w = c * NS + s
    pltpu.sync_copy(col_h.at[w], colv)
    pltpu.sync_copy(et_h.at[w], etv)
    pltpu.sync_copy(row_h.at[w], rowv)

    zero16 = jnp.zeros((16,), jnp.float32)

    def _z(i, carry):
        degv[pl.ds(i * 16, 16)] = zero16
        return carry

    lax.fori_loop(0, NPAD // 16, _z, 0)

    ones16 = jnp.ones((16,), jnp.float32)

    def _e(k, carry):
        cc = colv[pl.ds(k * 16, 16)]
        tt = etv[pl.ds(k * 16, 16)]
        rr = rowv[pl.ds(k * 16, 16)]
        gv[pl.ds(k * 16, 16)] = (tt * N + cc) | (rr << 16)
        plsc.addupdate_scatter(degv, [cc], ones16)
        return carry

    lax.fori_loop(0, EPT0 // 16, _e, 0)

    pltpu.sync_copy(gv, gidx_h.at[w])
    pltpu.sync_copy(degv, degp_h.at[w])


def _sck0(col2, et2, row2):
    return pl.kernel(
        _sck0_body,
        out_type=(
            jax.ShapeDtypeStruct((NW, EPT0), jnp.int32),
            jax.ShapeDtypeStruct((NW, NPAD), jnp.float32),
        ),
        mesh=_mesh(),
        scratch_types=[
            pltpu.VMEM((EPT0,), jnp.int32),
            pltpu.VMEM((EPT0,), jnp.int32),
            pltpu.VMEM((EPT0,), jnp.int32),
            pltpu.VMEM((EPT0,), jnp.int32),
            pltpu.VMEM((NPAD,), jnp.float32),
        ],
        **_SC_PARAMS,
    )(col2, et2, row2)


def _sck_edge_body(hp_h, pk_h, attr_h, p_h,
                   pkv, attrv, gixa, gixb, rwa, rwb,
                   gbufa, gbufb, acc, sema, semb):
    c = lax.axis_index("c")
    s = lax.axis_index("s")
    gb = [gbufa, gbufb]
    gix = [gixa, gixb]
    rw = [rwa, rwb]
    gsem = [sema, semb]

    @pl.when(c == 0)
    def _stage_a():
        eb = s * (EPB_A * CHUNK)
        pltpu.sync_copy(pk_h.at[pl.ds(eb, EPB_A * CHUNK)], pkv.at[pl.ds(0, EPB_A * CHUNK)])
        pltpu.sync_copy(attr_h.at[pl.ds(eb, EPB_A * CHUNK)], attrv.at[pl.ds(0, EPB_A * CHUNK)])

    @pl.when(c == 1)
    def _stage_b():
        eb = NS * (EPB_A * CHUNK) + s * (EPB_B * CHUNK)
        pltpu.sync_copy(pk_h.at[pl.ds(eb, EPB_B * CHUNK)], pkv.at[pl.ds(0, EPB_B * CHUNK)])
        pltpu.sync_copy(attr_h.at[pl.ds(eb, EPB_B * CHUNK)], attrv.at[pl.ds(0, EPB_B * CHUNK)])

    nmine = jnp.where(c == 0, EPB_A, EPB_B)

    zero16 = jnp.zeros((16,), jnp.float32)

    def _zrow(i, carry):
        for q in range(D // 16):
            gbufa[i, pl.ds(q * 16, 16)] = zero16
        return carry

    lax.fori_loop(0, CHUNK, _zrow, 0)

    base = s * RPT
    for k in range(RPT // CHUNK):
        pltpu.sync_copy(gbufa, acc.at[pl.ds(base + k * CHUNK, CHUNK)])
    rem = RPT % CHUNK
    if rem:
        pltpu.sync_copy(gbufa.at[pl.ds(0, rem)],
                        acc.at[pl.ds(base + RPT - rem, rem)])

    mask16 = jnp.full((16,), 0xFFFF, jnp.int32)

    def _unpack(j, slot):
        def _u(k, carry):
            pk = pkv[pl.ds(j * CHUNK + k * 16, 16)]
            gix[slot][pl.ds(k * 16, 16)] = pk & mask16
            rw[slot][pl.ds(k * 16, 16)] = lax.shift_right_logical(pk, 16)
            return carry
        lax.fori_loop(0, CHUNK // 16, _u, 0)

    _unpack(0, 0)
    pltpu.async_copy(hp_h.at[gixa], gbufa, gsem[0])
    plsc.subcore_barrier()

    def _iter(j, b):
        pltpu.make_async_copy(hp_h.at[gix[b]], gb[b], gsem[b]).wait()

        @pl.when(j + 1 < nmine)
        def _next():
            _unpack(j + 1, 1 - b)
            pltpu.async_copy(hp_h.at[gix[1 - b]], gb[1 - b], gsem[1 - b])

        def _srow(i, carry2):
            a = plsc.load_gather(attrv, [jnp.full((16,), j * CHUNK + i, jnp.int32)])
            for q in range(D // 16):
                gb[b][i, pl.ds(q * 16, 16)] = gb[b][i, pl.ds(q * 16, 16)] * a
            return carry2

        lax.fori_loop(0, CHUNK, _srow, 0, unroll=4)
        pltpu.sync_copy(gb[b], acc.at[rw[b]], add=True)

    def _outer(t, carry):
        for b in range(2):
            _iter(t * 2 + b, b)
        return carry

    lax.fori_loop(0, nmine // 2, _outer, 0)

    plsc.subcore_barrier()
    pltpu.sync_copy(acc.at[pl.ds(base, RPT)], p_h.at[c, pl.ds(base, RPT)])


def _sck_edge(hp_flat, pk, attr):
    return pl.kernel(
        _sck_edge_body,
        out_type=jax.ShapeDtypeStruct((NC, N, D), jnp.float32),
        mesh=_mesh(),
        scratch_types=[
            pltpu.VMEM((EPB_A * CHUNK,), jnp.int32),
            pltpu.VMEM((EPB_A * CHUNK,), jnp.float32),
            pltpu.VMEM((CHUNK,), jnp.int32),
            pltpu.VMEM((CHUNK,), jnp.int32),
            pltpu.VMEM((CHUNK,), jnp.int32),
            pltpu.VMEM((CHUNK,), jnp.int32),
            pltpu.VMEM((CHUNK, D), jnp.float32),
            pltpu.VMEM((CHUNK, D), jnp.float32),
            pltpu.VMEM_SHARED((N, D), jnp.float32),
            pltpu.SemaphoreType.DMA,
            pltpu.SemaphoreType.DMA,
        ],
        **_SC_PARAMS,
    )(hp_flat, pk, attr)


def _tck0_body(degp_ref, dis_ref):
    deg = jnp.sum(degp_ref[...], axis=0)
    dis = jnp.where(deg > 0, lax.rsqrt(deg), 0.0)
    dis_ref[...] = dis[:, None]


def _tck0(degp):
    return pl.pallas_call(
        _tck0_body,
        out_shape=jax.ShapeDtypeStruct((NPAD, 1), jnp.float32),
    )(degp)


def _tck1_body(x_ref, dis_ref, w_ref, hp_ref):
    xs = x_ref[...] * dis_ref[...]
    for r in range(R):
        hp_ref[r, :, :] = jnp.dot(xs, w_ref[r], preferred_element_type=jnp.float32)


def _tck1(x, dis, w):
    return pl.pallas_call(
        _tck1_body,
        grid=(N // _BLK,),
        in_specs=[
            pl.BlockSpec((_BLK, D), lambda i: (i, 0)),
            pl.BlockSpec((_BLK, 1), lambda i: (i, 0)),
            pl.BlockSpec((R, D, D), lambda i: (0, 0, 0)),
        ],
        out_specs=pl.BlockSpec((R, _BLK, D), lambda i: (0, i, 0)),
        out_shape=jax.ShapeDtypeStruct((R, N, D), jnp.float32),
    )(x, dis, w)


def _tck_mid_body(p_ref, dis_ref, b_ref, w_ref, z_ref, hp_ref):
    dis = dis_ref[...]
    v = (p_ref[0] + p_ref[1]) * dis + b_ref[...]
    z = jnp.where(v >= 0, v, 0.01 * v)
    z_ref[...] = z
    zs = z * dis
    for r in range(R):
        hp_ref[r, :, :] = jnp.dot(zs, w_ref[r], preferred_element_type=jnp.float32)


def _tck_mid(p, dis, b, w):
    return pl.pallas_call(
        _tck_mid_body,
        grid=(N // _BLK,),
        in_specs=[
            pl.BlockSpec((NC, _BLK, D), lambda i: (0, i, 0)),
            pl.BlockSpec((_BLK, 1), lambda i: (i, 0)),
            pl.BlockSpec((1, D), lambda i: (0, 0)),
            pl.BlockSpec((R, D, D), lambda i: (0, 0, 0)),
        ],
        out_specs=[
            pl.BlockSpec((_BLK, D), lambda i: (i, 0)),
            pl.BlockSpec((R, _BLK, D), lambda i: (0, i, 0)),
        ],
        out_shape=[
            jax.ShapeDtypeStruct((N, D), jnp.float32),
            jax.ShapeDtypeStruct((R, N, D), jnp.float32),
        ],
    )(p, dis, b, w)


def _tck_fin_body(p_ref, dis_ref, b_ref, x_ref, z1_ref, z2_ref, z_ref):
    v = (p_ref[0] + p_ref[1]) * dis_ref[...] + b_ref[...]
    z3 = jnp.where(v >= 0, v, 0.01 * v)
    z_ref[...] = (x_ref[...] + z1_ref[...] + z2_ref[...] + z3) * 0.25


def _tck_fin(p, dis, b, x, z1, z2):
    return pl.pallas_call(
        _tck_fin_body,
        grid=(N // _BLK,),
        in_specs=[
            pl.BlockSpec((NC, _BLK, D), lambda i: (0, i, 0)),
            pl.BlockSpec((_BLK, 1), lambda i: (i, 0)),
            pl.BlockSpec((1, D), lambda i: (0, 0)),
            pl.BlockSpec((_BLK, D), lambda i: (i, 0)),
            pl.BlockSpec((_BLK, D), lambda i: (i, 0)),
            pl.BlockSpec((_BLK, D), lambda i: (i, 0)),
        ],
        out_specs=pl.BlockSpec((_BLK, D), lambda i: (i, 0)),
        out_shape=jax.ShapeDtypeStruct((N, D), jnp.float32),
    )(p, dis, b, x, z1, z2)


def kernel(x, edge_index, edge_type, edge_attr, w1, b1, w2, b2, w3, b3):
    row2 = edge_index[0].reshape(NW, EPT0)
    col2 = edge_index[1].reshape(NW, EPT0)
    et2 = edge_type.reshape(NW, EPT0)

    pk2, degp = _sck0(col2, et2, row2)
    pk = pk2.reshape(E)

    dis = _tck0(degp)

    hp1 = _tck1(x, dis, w1)
    p1 = _sck_edge(hp1.reshape(R * N, D), pk, edge_attr)
    z1, hp2 = _tck_mid(p1, dis, b1.reshape(1, D), w2)
    p2 = _sck_edge(hp2.reshape(R * N, D), pk, edge_attr)
    z2, hp3 = _tck_mid(p2, dis, b2.reshape(1, D), w3)
    p3 = _sck_edge(hp3.reshape(R * N, D), pk, edge_attr)
    z = _tck_fin(p3, dis, b3.reshape(1, D), x, z1, z2)
    return z
